# Initial kernel scaffold; baseline (speedup 1.0000x reference)
import jax
import jax.numpy as jnp
from jax.experimental import pallas as pl


def kernel(partial_cloud, W1, b1, W2, b2, W3, b3, W4, b4, W5, b5, W6, b6, W7, b7, W8, b8, W9, b9):
    raise NotImplementedError("write your pallas kernel here")


# speedup vs baseline: 3.4107x; 3.4107x over previous
import functools

import jax
import jax.numpy as jnp
from jax.experimental import pallas as pl
from jax.experimental.pallas import tpu as pltpu

M_LATTICE = 8192
P1, P2, P3 = 73856093, 19349663, 83492791
SCALES = (64.0, 32.0, 16.0, 8.0, 4.0)
EPS = 1e-5


def _bn_relu_from_stats(x, stats_ref):
    mean = stats_ref[0:1, :]
    var = stats_ref[1:2, :]
    return jnp.maximum((x - mean) / jnp.sqrt(var + EPS), 0.0)


def _accum_stats(i, n_total, r, sums_ref, stats_out_ref, num_programs):
    @pl.when(i == 0)
    def _init():
        sums_ref[:, :] = jnp.zeros_like(sums_ref)

    sums_ref[0:1, :] += jnp.sum(r, axis=0, keepdims=True)
    sums_ref[1:2, :] += jnp.sum(r * r, axis=0, keepdims=True)

    @pl.when(i == num_programs - 1)
    def _emit():
        m = sums_ref[0:1, :] / n_total
        v = sums_ref[1:2, :] / n_total - m * m
        stats_out_ref[0:1, :] = m
        stats_out_ref[1:2, :] = v


def _k1_body(pc_ref, w_ref, b_ref, raw_ref, stats_ref, keys_ref):
    B, N, _ = pc_ref.shape
    x = pc_ref[:, :, :]
    xr = x.reshape(B * N, 3)
    r = jnp.dot(xr, w_ref[:, :], preferred_element_type=jnp.float32) + b_ref[0:1, :]
    raw_ref[:, :, :] = r.reshape(B, N, -1)
    m = jnp.mean(r, axis=0, keepdims=True)
    v = jnp.mean((r - m) ** 2, axis=0, keepdims=True)
    stats_ref[0:1, :] = m
    stats_ref[1:2, :] = v
    for i, s in enumerate(SCALES):
        ip = jnp.floor(x * s).astype(jnp.int32)
        h = ip[:, :, 0] * P1 + ip[:, :, 1] * P2 + ip[:, :, 2] * P3
        keys_ref[i, :, :] = jnp.bitwise_and(h, M_LATTICE - 1)


def _stage1(pc, W1, b1):
    B, N, _ = pc.shape
    C = W1.shape[1]
    return pl.pallas_call(
        _k1_body,
        out_shape=[
            jax.ShapeDtypeStruct((B, N, C), jnp.float32),
            jax.ShapeDtypeStruct((2, C), jnp.float32),
            jax.ShapeDtypeStruct((len(SCALES), B, N), jnp.int32),
        ],
    )(pc, W1, b1)


def _perm_body(has_acc, raw_ref, stats_ref, key_ref, w_ref, b_ref, *rest):
    if has_acc:
        (acc_ref, w7_ref, out_ref, stats_out_ref, acc_out_ref, sums_ref) = rest
    else:
        (out_ref, stats_out_ref, sums_ref) = rest
    i = pl.program_id(0)
    B = pl.num_programs(0)
    _, N, _ = raw_ref.shape
    f = _bn_relu_from_stats(raw_ref[0], stats_ref)
    k = key_ref[0]
    A = (k.reshape(N, 1) == k.reshape(1, N)).astype(jnp.float32)
    cnt = jnp.sum(A, axis=1, keepdims=True)
    s = jnp.dot(A, f, preferred_element_type=jnp.float32, precision=jax.lax.Precision.HIGHEST)
    sliced = s / cnt
    r = jnp.dot(sliced, w_ref[:, :], preferred_element_type=jnp.float32) + b_ref[0:1, :]
    out_ref[0] = r
    if has_acc:
        acc_out_ref[0] = acc_ref[0] + jnp.dot(
            f, w7_ref[:, :], preferred_element_type=jnp.float32)
    _accum_stats(i, B * N, r, sums_ref, stats_out_ref, B)


def _perm_stage(raw_prev, stats_prev, keys_l, W, bvec, acc_in=None, W7c=None):
    B, N, Cin = raw_prev.shape
    Cout = W.shape[1]
    has_acc = acc_in is not None
    full = lambda shape: pl.BlockSpec(shape, lambda b: (0,) * len(shape))
    per_b = lambda shape: pl.BlockSpec((1,) + shape, lambda b: (b,) + (0,) * len(shape))
    in_specs = [
        per_b((N, Cin)),
        full((2, Cin)),
        per_b((1, N)),
        full((Cin, Cout)),
        full((1, Cout)),
    ]
    out_shape = [
        jax.ShapeDtypeStruct((B, N, Cout), jnp.float32),
        jax.ShapeDtypeStruct((2, Cout), jnp.float32),
    ]
    out_specs = [per_b((N, Cout)), full((2, Cout))]
    args = [raw_prev, stats_prev, keys_l, W, bvec]
    if has_acc:
        Cacc = acc_in.shape[2]
        in_specs += [per_b((N, Cacc)), full((Cin, Cacc))]
        out_shape.append(jax.ShapeDtypeStruct((B, N, Cacc), jnp.float32))
        out_specs.append(per_b((N, Cacc)))
        args += [acc_in, W7c]
    return pl.pallas_call(
        functools.partial(_perm_body, has_acc),
        grid=(B,),
        in_specs=in_specs,
        out_specs=out_specs,
        out_shape=out_shape,
        scratch_shapes=[pltpu.VMEM((2, Cout), jnp.float32)],
    )(*args)


def _conv_body(has_acc, raw_ref, stats_ref, w_ref, b_ref, *rest):
    if has_acc:
        (acc_ref, out_ref, stats_out_ref, sums_ref) = rest
    else:
        (out_ref, stats_out_ref, sums_ref) = rest
    i = pl.program_id(0)
    B = pl.num_programs(0)
    _, N, _ = raw_ref.shape
    f = _bn_relu_from_stats(raw_ref[0], stats_ref)
    r = jnp.dot(f, w_ref[:, :], preferred_element_type=jnp.float32) + b_ref[0:1, :]
    if has_acc:
        r = r + acc_ref[0]
    out_ref[0] = r
    _accum_stats(i, B * N, r, sums_ref, stats_out_ref, B)


def _conv_stage(raw_prev, stats_prev, W, bvec, acc_in=None):
    B, N, Cin = raw_prev.shape
    Cout = W.shape[1]
    has_acc = acc_in is not None
    full = lambda shape: pl.BlockSpec(shape, lambda b: (0,) * len(shape))
    per_b = lambda shape: pl.BlockSpec((1,) + shape, lambda b: (b,) + (0,) * len(shape))
    in_specs = [per_b((N, Cin)), full((2, Cin)), full((Cin, Cout)), full((1, Cout))]
    args = [raw_prev, stats_prev, W, bvec]
    if has_acc:
        in_specs.append(per_b((N, Cout)))
        args.append(acc_in)
    return pl.pallas_call(
        functools.partial(_conv_body, has_acc),
        grid=(B,),
        in_specs=in_specs,
        out_specs=[per_b((N, Cout)), full((2, Cout))],
        out_shape=[
            jax.ShapeDtypeStruct((B, N, Cout), jnp.float32),
            jax.ShapeDtypeStruct((2, Cout), jnp.float32),
        ],
        scratch_shapes=[pltpu.VMEM((2, Cout), jnp.float32)],
    )(*args)


def _final_body(raw_ref, stats_ref, out_ref):
    B, N, C = raw_ref.shape
    x = raw_ref[:, :, :].reshape(B * N, C)
    y = _bn_relu_from_stats(x, stats_ref)
    out_ref[:, :, :] = y.reshape(B, N, C)


def _final_stage(raw, stats):
    B, N, C = raw.shape
    return pl.pallas_call(
        _final_body,
        out_shape=jax.ShapeDtypeStruct((B, N, C), jnp.float32),
    )(raw, stats)


def kernel(partial_cloud, W1, b1, W2, b2, W3, b3, W4, b4, W5, b5, W6, b6,
           W7, b7, W8, b8, W9, b9):
    B, N, _ = partial_cloud.shape
    row = lambda v: v.reshape(1, -1)

    raw1, stats1, keys = _stage1(partial_cloud, W1, row(b1))
    keys = keys.reshape(len(SCALES), B, 1, N)

    raw2, stats2 = _perm_stage(raw1, stats1, keys[0], W2, row(b2))
    acc0 = jnp.zeros((B, N, W7.shape[1]), jnp.float32)
    offs = [0, 64, 192, 448, 704, 960]
    raw3, stats3, acc = _perm_stage(raw2, stats2, keys[1], W3, row(b3),
                                    acc0, W7[offs[0]:offs[1]])
    raw4, stats4, acc = _perm_stage(raw3, stats3, keys[2], W4, row(b4),
                                    acc, W7[offs[1]:offs[2]])
    raw5, stats5, acc = _perm_stage(raw4, stats4, keys[3], W5, row(b5),
                                    acc, W7[offs[2]:offs[3]])
    raw6, stats6, acc = _perm_stage(raw5, stats5, keys[4], W6, row(b6),
                                    acc, W7[offs[3]:offs[4]])
    raw7, stats7 = _conv_stage(raw6, stats6, W7[offs[4]:offs[5]], row(b7), acc)
    raw8, stats8 = _conv_stage(raw7, stats7, W8, row(b8))
    raw9, stats9 = _conv_stage(raw8, stats8, W9, row(b9))
    out = _final_stage(raw9, stats9)
    return jnp.transpose(out, (0, 2, 1))


# speedup vs baseline: 5.8234x; 1.7074x over previous
import functools

import jax
import jax.numpy as jnp
from jax.experimental import pallas as pl
from jax.experimental.pallas import tpu as pltpu

M_LATTICE = 8192
P1, P2, P3 = 73856093, 19349663, 83492791
SCALES = (64.0, 32.0, 16.0, 8.0, 4.0)
EPS = 1e-5


def _bn_relu_from_stats(x, stats_ref):
    mean = stats_ref[0:1, :]
    var = stats_ref[1:2, :]
    return jnp.maximum((x - mean) / jnp.sqrt(var + EPS), 0.0)


def _accum_stats(i, n_total, r, sums_ref, stats_out_ref, num_programs):
    @pl.when(i == 0)
    def _init():
        sums_ref[:, :] = jnp.zeros_like(sums_ref)

    sums_ref[0:1, :] += jnp.sum(r, axis=0, keepdims=True)
    sums_ref[1:2, :] += jnp.sum(r * r, axis=0, keepdims=True)

    @pl.when(i == num_programs - 1)
    def _emit():
        m = sums_ref[0:1, :] / n_total
        v = sums_ref[1:2, :] / n_total - m * m
        stats_out_ref[0:1, :] = m
        stats_out_ref[1:2, :] = v


def _k1_body(pc_ref, w_ref, b_ref, raw_ref, stats_ref, keys_ref):
    B, N, _ = pc_ref.shape
    x = pc_ref[:, :, :]
    xr = x.reshape(B * N, 3)
    r = jnp.dot(xr, w_ref[:, :], preferred_element_type=jnp.float32) + b_ref[0:1, :]
    raw_ref[:, :, :] = r.reshape(B, N, -1)
    m = jnp.mean(r, axis=0, keepdims=True)
    v = jnp.mean((r - m) ** 2, axis=0, keepdims=True)
    stats_ref[0:1, :] = m
    stats_ref[1:2, :] = v
    for i, s in enumerate(SCALES):
        ip = jnp.floor(x * s).astype(jnp.int32)
        h = ip[:, :, 0] * P1 + ip[:, :, 1] * P2 + ip[:, :, 2] * P3
        keys_ref[i, :, :] = jnp.bitwise_and(h, M_LATTICE - 1)


def _stage1(pc, W1, b1):
    B, N, _ = pc.shape
    C = W1.shape[1]
    return pl.pallas_call(
        _k1_body,
        out_shape=[
            jax.ShapeDtypeStruct((B, N, C), jnp.float32),
            jax.ShapeDtypeStruct((2, C), jnp.float32),
            jax.ShapeDtypeStruct((len(SCALES), B, N), jnp.int32),
        ],
    )(pc, W1, b1)


def _perm_body(has_acc, raw_ref, stats_ref, key_ref, w_ref, b_ref, *rest):
    if has_acc:
        (acc_ref, w7_ref, out_ref, stats_out_ref, acc_out_ref, sums_ref) = rest
    else:
        (out_ref, stats_out_ref, sums_ref) = rest
    i = pl.program_id(0)
    B = pl.num_programs(0)
    _, N, _ = raw_ref.shape
    f = _bn_relu_from_stats(raw_ref[0], stats_ref)
    k = key_ref[0]
    A = (k.reshape(N, 1) == k.reshape(1, N)).astype(jnp.bfloat16)
    f_hi = f.astype(jnp.bfloat16)
    f_lo = (f - f_hi.astype(jnp.float32)).astype(jnp.bfloat16)
    s = (jnp.dot(A, f_hi, preferred_element_type=jnp.float32)
         + jnp.dot(A, f_lo, preferred_element_type=jnp.float32))
    cnt = jnp.dot(A, jnp.ones((N, 1), jnp.bfloat16),
                  preferred_element_type=jnp.float32)
    sliced = s / cnt
    r = jnp.dot(sliced, w_ref[:, :], preferred_element_type=jnp.float32) + b_ref[0:1, :]
    out_ref[0] = r
    if has_acc:
        acc_out_ref[0] = acc_ref[0] + jnp.dot(
            f, w7_ref[:, :], preferred_element_type=jnp.float32)
    _accum_stats(i, B * N, r, sums_ref, stats_out_ref, B)


def _perm_stage(raw_prev, stats_prev, keys_l, W, bvec, acc_in=None, W7c=None):
    B, N, Cin = raw_prev.shape
    Cout = W.shape[1]
    has_acc = acc_in is not None
    full = lambda shape: pl.BlockSpec(shape, lambda b: (0,) * len(shape))
    per_b = lambda shape: pl.BlockSpec((1,) + shape, lambda b: (b,) + (0,) * len(shape))
    in_specs = [
        per_b((N, Cin)),
        full((2, Cin)),
        per_b((1, N)),
        full((Cin, Cout)),
        full((1, Cout)),
    ]
    out_shape = [
        jax.ShapeDtypeStruct((B, N, Cout), jnp.float32),
        jax.ShapeDtypeStruct((2, Cout), jnp.float32),
    ]
    out_specs = [per_b((N, Cout)), full((2, Cout))]
    args = [raw_prev, stats_prev, keys_l, W, bvec]
    if has_acc:
        Cacc = acc_in.shape[2]
        in_specs += [per_b((N, Cacc)), full((Cin, Cacc))]
        out_shape.append(jax.ShapeDtypeStruct((B, N, Cacc), jnp.float32))
        out_specs.append(per_b((N, Cacc)))
        args += [acc_in, W7c]
    return pl.pallas_call(
        functools.partial(_perm_body, has_acc),
        grid=(B,),
        in_specs=in_specs,
        out_specs=out_specs,
        out_shape=out_shape,
        scratch_shapes=[pltpu.VMEM((2, Cout), jnp.float32)],
    )(*args)


def _conv_body(has_acc, raw_ref, stats_ref, w_ref, b_ref, *rest):
    if has_acc:
        (acc_ref, out_ref, stats_out_ref, sums_ref) = rest
    else:
        (out_ref, stats_out_ref, sums_ref) = rest
    i = pl.program_id(0)
    B = pl.num_programs(0)
    _, N, _ = raw_ref.shape
    f = _bn_relu_from_stats(raw_ref[0], stats_ref)
    r = jnp.dot(f, w_ref[:, :], preferred_element_type=jnp.float32) + b_ref[0:1, :]
    if has_acc:
        r = r + acc_ref[0]
    out_ref[0] = r
    _accum_stats(i, B * N, r, sums_ref, stats_out_ref, B)


def _conv_stage(raw_prev, stats_prev, W, bvec, acc_in=None):
    B, N, Cin = raw_prev.shape
    Cout = W.shape[1]
    has_acc = acc_in is not None
    full = lambda shape: pl.BlockSpec(shape, lambda b: (0,) * len(shape))
    per_b = lambda shape: pl.BlockSpec((1,) + shape, lambda b: (b,) + (0,) * len(shape))
    in_specs = [per_b((N, Cin)), full((2, Cin)), full((Cin, Cout)), full((1, Cout))]
    args = [raw_prev, stats_prev, W, bvec]
    if has_acc:
        in_specs.append(per_b((N, Cout)))
        args.append(acc_in)
    return pl.pallas_call(
        functools.partial(_conv_body, has_acc),
        grid=(B,),
        in_specs=in_specs,
        out_specs=[per_b((N, Cout)), full((2, Cout))],
        out_shape=[
            jax.ShapeDtypeStruct((B, N, Cout), jnp.float32),
            jax.ShapeDtypeStruct((2, Cout), jnp.float32),
        ],
        scratch_shapes=[pltpu.VMEM((2, Cout), jnp.float32)],
    )(*args)


def _final_body(raw_ref, stats_ref, out_ref):
    B, N, C = raw_ref.shape
    x = raw_ref[:, :, :].reshape(B * N, C)
    y = _bn_relu_from_stats(x, stats_ref)
    out_ref[:, :, :] = y.reshape(B, N, C)


def _final_stage(raw, stats):
    B, N, C = raw.shape
    return pl.pallas_call(
        _final_body,
        out_shape=jax.ShapeDtypeStruct((B, N, C), jnp.float32),
    )(raw, stats)


def kernel(partial_cloud, W1, b1, W2, b2, W3, b3, W4, b4, W5, b5, W6, b6,
           W7, b7, W8, b8, W9, b9):
    B, N, _ = partial_cloud.shape
    row = lambda v: v.reshape(1, -1)

    raw1, stats1, keys = _stage1(partial_cloud, W1, row(b1))
    keys = keys.reshape(len(SCALES), B, 1, N)

    raw2, stats2 = _perm_stage(raw1, stats1, keys[0], W2, row(b2))
    acc0 = jnp.zeros((B, N, W7.shape[1]), jnp.float32)
    offs = [0, 64, 192, 448, 704, 960]
    raw3, stats3, acc = _perm_stage(raw2, stats2, keys[1], W3, row(b3),
                                    acc0, W7[offs[0]:offs[1]])
    raw4, stats4, acc = _perm_stage(raw3, stats3, keys[2], W4, row(b4),
                                    acc, W7[offs[1]:offs[2]])
    raw5, stats5, acc = _perm_stage(raw4, stats4, keys[3], W5, row(b5),
                                    acc, W7[offs[2]:offs[3]])
    raw6, stats6, acc = _perm_stage(raw5, stats5, keys[4], W6, row(b6),
                                    acc, W7[offs[3]:offs[4]])
    raw7, stats7 = _conv_stage(raw6, stats6, W7[offs[4]:offs[5]], row(b7), acc)
    raw8, stats8 = _conv_stage(raw7, stats7, W8, row(b8))
    raw9, stats9 = _conv_stage(raw8, stats8, W9, row(b9))
    out = _final_stage(raw9, stats9)
    return jnp.transpose(out, (0, 2, 1))


# speedup vs baseline: 6.5602x; 1.1265x over previous
import functools

import jax
import jax.numpy as jnp
from jax.experimental import pallas as pl
from jax.experimental.pallas import tpu as pltpu

M_LATTICE = 8192
P1, P2, P3 = 73856093, 19349663, 83492791
SCALES = (64.0, 32.0, 16.0, 8.0, 4.0)
EPS = 1e-5


def _bn_relu_from_stats(x, stats_ref):
    mean = stats_ref[0:1, :]
    var = stats_ref[1:2, :]
    return jnp.maximum((x - mean) / jnp.sqrt(var + EPS), 0.0)


def _accum_stats(i, n_total, r, sums_ref, stats_out_ref, num_programs):
    @pl.when(i == 0)
    def _init():
        sums_ref[:, :] = jnp.zeros_like(sums_ref)

    sums_ref[0:1, :] += jnp.sum(r, axis=0, keepdims=True)
    sums_ref[1:2, :] += jnp.sum(r * r, axis=0, keepdims=True)

    @pl.when(i == num_programs - 1)
    def _emit():
        m = sums_ref[0:1, :] / n_total
        v = sums_ref[1:2, :] / n_total - m * m
        stats_out_ref[0:1, :] = m
        stats_out_ref[1:2, :] = v


def _k1_body(pc_ref, w_ref, b_ref, raw_ref, stats_ref, keys_ref):
    B, N, _ = pc_ref.shape
    x = pc_ref[:, :, :]
    xr = x.reshape(B * N, 3)
    r = jnp.dot(xr, w_ref[:, :], preferred_element_type=jnp.float32) + b_ref[0:1, :]
    raw_ref[:, :, :] = r.reshape(B, N, -1)
    m = jnp.mean(r, axis=0, keepdims=True)
    v = jnp.mean((r - m) ** 2, axis=0, keepdims=True)
    stats_ref[0:1, :] = m
    stats_ref[1:2, :] = v
    for i, s in enumerate(SCALES):
        ip = jnp.floor(x * s).astype(jnp.int32)
        h = ip[:, :, 0] * P1 + ip[:, :, 1] * P2 + ip[:, :, 2] * P3
        keys_ref[i, :, :] = jnp.bitwise_and(h, M_LATTICE - 1)


def _stage1(pc, W1, b1):
    B, N, _ = pc.shape
    C = W1.shape[1]
    return pl.pallas_call(
        _k1_body,
        out_shape=[
            jax.ShapeDtypeStruct((B, N, C), jnp.float32),
            jax.ShapeDtypeStruct((2, C), jnp.float32),
            jax.ShapeDtypeStruct((len(SCALES), B, N), jnp.int32),
        ],
    )(pc, W1, b1)


def _perm_body(acc_mode, raw_ref, stats_ref, key_ref, w_ref, b_ref, *rest):
    if acc_mode == 'add':
        (acc_ref, w7_ref, out_ref, stats_out_ref, acc_out_ref, sums_ref) = rest
    elif acc_mode == 'init':
        (w7_ref, out_ref, stats_out_ref, acc_out_ref, sums_ref) = rest
    else:
        (out_ref, stats_out_ref, sums_ref) = rest
    i = pl.program_id(0)
    B = pl.num_programs(0)
    _, N, Cin = raw_ref.shape
    f = _bn_relu_from_stats(raw_ref[0], stats_ref)
    k = key_ref[0]
    A = (k.reshape(N, 1) == k.reshape(1, N)).astype(jnp.bfloat16)
    f_hi = f.astype(jnp.bfloat16)
    f_lo = (f - f_hi.astype(jnp.float32)).astype(jnp.bfloat16)
    if Cin % 128 != 0:
        f_hi = jnp.concatenate([f_hi, jnp.ones((N, 1), jnp.bfloat16)], axis=1)
        f_lo = jnp.concatenate([f_lo, jnp.zeros((N, 1), jnp.bfloat16)], axis=1)
        s_aug = (jnp.dot(A, f_hi, preferred_element_type=jnp.float32)
                 + jnp.dot(A, f_lo, preferred_element_type=jnp.float32))
        s = s_aug[:, :Cin]
        cnt = s_aug[:, Cin:Cin + 1]
    else:
        s = (jnp.dot(A, f_hi, preferred_element_type=jnp.float32)
             + jnp.dot(A, f_lo, preferred_element_type=jnp.float32))
        cnt = jnp.dot(A, jnp.ones((N, 1), jnp.bfloat16),
                      preferred_element_type=jnp.float32)
    sliced = s / cnt
    r = jnp.dot(sliced, w_ref[:, :], preferred_element_type=jnp.float32) + b_ref[0:1, :]
    out_ref[0] = r
    if acc_mode == 'add':
        acc_out_ref[0] = acc_ref[0] + jnp.dot(
            f, w7_ref[:, :], preferred_element_type=jnp.float32)
    elif acc_mode == 'init':
        acc_out_ref[0] = jnp.dot(
            f, w7_ref[:, :], preferred_element_type=jnp.float32)
    _accum_stats(i, B * N, r, sums_ref, stats_out_ref, B)


def _perm_stage(raw_prev, stats_prev, keys_l, W, bvec, acc_in=None, W7c=None):
    B, N, Cin = raw_prev.shape
    Cout = W.shape[1]
    if W7c is None:
        acc_mode = 'none'
    elif acc_in is None:
        acc_mode = 'init'
    else:
        acc_mode = 'add'
    full = lambda shape: pl.BlockSpec(shape, lambda b: (0,) * len(shape))
    per_b = lambda shape: pl.BlockSpec((1,) + shape, lambda b: (b,) + (0,) * len(shape))
    in_specs = [
        per_b((N, Cin)),
        full((2, Cin)),
        per_b((1, N)),
        full((Cin, Cout)),
        full((1, Cout)),
    ]
    out_shape = [
        jax.ShapeDtypeStruct((B, N, Cout), jnp.float32),
        jax.ShapeDtypeStruct((2, Cout), jnp.float32),
    ]
    out_specs = [per_b((N, Cout)), full((2, Cout))]
    args = [raw_prev, stats_prev, keys_l, W, bvec]
    if acc_mode != 'none':
        Cacc = W7c.shape[1]
        if acc_mode == 'add':
            in_specs.append(per_b((N, Cacc)))
            args.append(acc_in)
        in_specs.append(full((Cin, Cacc)))
        args.append(W7c)
        out_shape.append(jax.ShapeDtypeStruct((B, N, Cacc), jnp.float32))
        out_specs.append(per_b((N, Cacc)))
    return pl.pallas_call(
        functools.partial(_perm_body, acc_mode),
        grid=(B,),
        in_specs=in_specs,
        out_specs=out_specs,
        out_shape=out_shape,
        scratch_shapes=[pltpu.VMEM((2, Cout), jnp.float32)],
    )(*args)


def _conv_body(has_acc, raw_ref, stats_ref, w_ref, b_ref, *rest):
    if has_acc:
        (acc_ref, out_ref, stats_out_ref, sums_ref) = rest
    else:
        (out_ref, stats_out_ref, sums_ref) = rest
    i = pl.program_id(0)
    B = pl.num_programs(0)
    _, N, _ = raw_ref.shape
    f = _bn_relu_from_stats(raw_ref[0], stats_ref)
    r = jnp.dot(f, w_ref[:, :], preferred_element_type=jnp.float32) + b_ref[0:1, :]
    if has_acc:
        r = r + acc_ref[0]
    out_ref[0] = r
    _accum_stats(i, B * N, r, sums_ref, stats_out_ref, B)


def _conv_stage(raw_prev, stats_prev, W, bvec, acc_in=None):
    B, N, Cin = raw_prev.shape
    Cout = W.shape[1]
    has_acc = acc_in is not None
    full = lambda shape: pl.BlockSpec(shape, lambda b: (0,) * len(shape))
    per_b = lambda shape: pl.BlockSpec((1,) + shape, lambda b: (b,) + (0,) * len(shape))
    in_specs = [per_b((N, Cin)), full((2, Cin)), full((Cin, Cout)), full((1, Cout))]
    args = [raw_prev, stats_prev, W, bvec]
    if has_acc:
        in_specs.append(per_b((N, Cout)))
        args.append(acc_in)
    return pl.pallas_call(
        functools.partial(_conv_body, has_acc),
        grid=(B,),
        in_specs=in_specs,
        out_specs=[per_b((N, Cout)), full((2, Cout))],
        out_shape=[
            jax.ShapeDtypeStruct((B, N, Cout), jnp.float32),
            jax.ShapeDtypeStruct((2, Cout), jnp.float32),
        ],
        scratch_shapes=[pltpu.VMEM((2, Cout), jnp.float32)],
    )(*args)


def _tail_body(raw_ref, stats_ref, w8_ref, b8_ref, w9_ref, b9_ref, out_ref):
    B, N, C = raw_ref.shape
    x = raw_ref[:, :, :].reshape(B * N, C)
    f7 = _bn_relu_from_stats(x, stats_ref)
    r8 = jnp.dot(f7, w8_ref[:, :], preferred_element_type=jnp.float32) + b8_ref[0:1, :]
    m8 = jnp.mean(r8, axis=0, keepdims=True)
    v8 = jnp.mean((r8 - m8) ** 2, axis=0, keepdims=True)
    f8 = jnp.maximum((r8 - m8) / jnp.sqrt(v8 + EPS), 0.0)
    r9 = jnp.dot(f8, w9_ref[:, :], preferred_element_type=jnp.float32) + b9_ref[0:1, :]
    m9 = jnp.mean(r9, axis=0, keepdims=True)
    v9 = jnp.mean((r9 - m9) ** 2, axis=0, keepdims=True)
    y = jnp.maximum((r9 - m9) / jnp.sqrt(v9 + EPS), 0.0)
    out_ref[:, :, :] = y.reshape(B, N, -1)


def _tail_stage(raw7, stats7, W8, b8, W9, b9):
    B, N, _ = raw7.shape
    return pl.pallas_call(
        _tail_body,
        out_shape=jax.ShapeDtypeStruct((B, N, W9.shape[1]), jnp.float32),
    )(raw7, stats7, W8, b8, W9, b9)


def kernel(partial_cloud, W1, b1, W2, b2, W3, b3, W4, b4, W5, b5, W6, b6,
           W7, b7, W8, b8, W9, b9):
    B, N, _ = partial_cloud.shape
    row = lambda v: v.reshape(1, -1)

    raw1, stats1, keys = _stage1(partial_cloud, W1, row(b1))
    keys = keys.reshape(len(SCALES), B, 1, N)

    raw2, stats2 = _perm_stage(raw1, stats1, keys[0], W2, row(b2))
    offs = [0, 64, 192, 448, 704, 960]
    raw3, stats3, acc = _perm_stage(raw2, stats2, keys[1], W3, row(b3),
                                    None, W7[offs[0]:offs[1]])
    raw4, stats4, acc = _perm_stage(raw3, stats3, keys[2], W4, row(b4),
                                    acc, W7[offs[1]:offs[2]])
    raw5, stats5, acc = _perm_stage(raw4, stats4, keys[3], W5, row(b5),
                                    acc, W7[offs[2]:offs[3]])
    raw6, stats6, acc = _perm_stage(raw5, stats5, keys[4], W6, row(b6),
                                    acc, W7[offs[3]:offs[4]])
    raw7, stats7 = _conv_stage(raw6, stats6, W7[offs[4]:offs[5]], row(b7), acc)
    out = _tail_stage(raw7, stats7, W8, row(b8), W9, row(b9))
    return jnp.transpose(out, (0, 2, 1))


# speedup vs baseline: 6.9083x; 1.0531x over previous
import functools

import jax
import jax.numpy as jnp
from jax.experimental import pallas as pl
from jax.experimental.pallas import tpu as pltpu

M_LATTICE = 8192
P1, P2, P3 = 73856093, 19349663, 83492791
SCALES = (64.0, 32.0, 16.0, 8.0, 4.0)
EPS = 1e-5


def _bn_relu_from_stats(x, stats_ref):
    mean = stats_ref[0:1, :]
    var = stats_ref[1:2, :]
    return jnp.maximum((x - mean) / jnp.sqrt(var + EPS), 0.0)


def _accum_stats(i, n_total, r, sums_ref, stats_out_ref, num_programs):
    @pl.when(i == 0)
    def _init():
        sums_ref[:, :] = jnp.zeros_like(sums_ref)

    sums_ref[0:1, :] += jnp.sum(r, axis=0, keepdims=True)
    sums_ref[1:2, :] += jnp.sum(r * r, axis=0, keepdims=True)

    @pl.when(i == num_programs - 1)
    def _emit():
        m = sums_ref[0:1, :] / n_total
        v = sums_ref[1:2, :] / n_total - m * m
        stats_out_ref[0:1, :] = m
        stats_out_ref[1:2, :] = v


def _k1_body(pc_ref, pct_ref, w_ref, b_ref, raw_ref, stats_ref, keys_ref):
    B, N, _ = pc_ref.shape
    xr = pc_ref[:, :, :].reshape(B * N, 3)
    r = jnp.dot(xr, w_ref[:, :], preferred_element_type=jnp.float32) + b_ref[0:1, :]
    raw_ref[:, :, :] = r.reshape(B, N, -1)
    m = jnp.mean(r, axis=0, keepdims=True)
    v = jnp.mean((r - m) ** 2, axis=0, keepdims=True)
    stats_ref[0:1, :] = m
    stats_ref[1:2, :] = v
    xt = pct_ref[:, :]
    for i, s in enumerate(SCALES):
        ip = jnp.floor(xt * s).astype(jnp.int32)
        h = ip[0:1, :] * P1 + ip[1:2, :] * P2 + ip[2:3, :] * P3
        keys_ref[i, :, :] = jnp.bitwise_and(h, M_LATTICE - 1)


def _stage1(pc, W1, b1):
    B, N, _ = pc.shape
    C = W1.shape[1]
    pct = pc.reshape(B * N, 3).T
    return pl.pallas_call(
        _k1_body,
        out_shape=[
            jax.ShapeDtypeStruct((B, N, C), jnp.float32),
            jax.ShapeDtypeStruct((2, C), jnp.float32),
            jax.ShapeDtypeStruct((len(SCALES), 1, B * N), jnp.int32),
        ],
    )(pc, pct, W1, b1)


def _perm_body(acc_mode, raw_ref, stats_ref, key_ref, w_ref, b_ref, *rest):
    if acc_mode == 'add':
        (acc_ref, w7_ref, out_ref, stats_out_ref, acc_out_ref, sums_ref) = rest
    elif acc_mode == 'init':
        (w7_ref, out_ref, stats_out_ref, acc_out_ref, sums_ref) = rest
    else:
        (out_ref, stats_out_ref, sums_ref) = rest
    i = pl.program_id(0)
    B = pl.num_programs(0)
    _, N, Cin = raw_ref.shape
    f = _bn_relu_from_stats(raw_ref[0], stats_ref)
    k = key_ref[0]
    A = (k.reshape(N, 1) == k.reshape(1, N)).astype(jnp.bfloat16)
    f_hi = f.astype(jnp.bfloat16)
    f_lo = (f - f_hi.astype(jnp.float32)).astype(jnp.bfloat16)
    if Cin % 128 != 0:
        f_hi = jnp.concatenate([f_hi, jnp.ones((N, 1), jnp.bfloat16)], axis=1)
        f_lo = jnp.concatenate([f_lo, jnp.zeros((N, 1), jnp.bfloat16)], axis=1)
        s_aug = (jnp.dot(A, f_hi, preferred_element_type=jnp.float32)
                 + jnp.dot(A, f_lo, preferred_element_type=jnp.float32))
        s = s_aug[:, :Cin]
        cnt = s_aug[:, Cin:Cin + 1]
    else:
        s = (jnp.dot(A, f_hi, preferred_element_type=jnp.float32)
             + jnp.dot(A, f_lo, preferred_element_type=jnp.float32))
        cnt = jnp.dot(A, jnp.ones((N, 1), jnp.bfloat16),
                      preferred_element_type=jnp.float32)
    sliced = s / cnt
    r = jnp.dot(sliced, w_ref[:, :], preferred_element_type=jnp.float32) + b_ref[0:1, :]
    out_ref[0] = r
    if acc_mode == 'add':
        acc_out_ref[0] = acc_ref[0] + jnp.dot(
            f, w7_ref[:, :], preferred_element_type=jnp.float32)
    elif acc_mode == 'init':
        acc_out_ref[0] = jnp.dot(
            f, w7_ref[:, :], preferred_element_type=jnp.float32)
    _accum_stats(i, B * N, r, sums_ref, stats_out_ref, B)


def _perm_stage(raw_prev, stats_prev, keys_l, W, bvec, acc_in=None, W7c=None):
    B, N, Cin = raw_prev.shape
    Cout = W.shape[1]
    if W7c is None:
        acc_mode = 'none'
    elif acc_in is None:
        acc_mode = 'init'
    else:
        acc_mode = 'add'
    full = lambda shape: pl.BlockSpec(shape, lambda b: (0,) * len(shape))
    per_b = lambda shape: pl.BlockSpec((1,) + shape, lambda b: (b,) + (0,) * len(shape))
    in_specs = [
        per_b((N, Cin)),
        full((2, Cin)),
        per_b((1, N)),
        full((Cin, Cout)),
        full((1, Cout)),
    ]
    out_shape = [
        jax.ShapeDtypeStruct((B, N, Cout), jnp.float32),
        jax.ShapeDtypeStruct((2, Cout), jnp.float32),
    ]
    out_specs = [per_b((N, Cout)), full((2, Cout))]
    args = [raw_prev, stats_prev, keys_l, W, bvec]
    if acc_mode != 'none':
        Cacc = W7c.shape[1]
        if acc_mode == 'add':
            in_specs.append(per_b((N, Cacc)))
            args.append(acc_in)
        in_specs.append(full((Cin, Cacc)))
        args.append(W7c)
        out_shape.append(jax.ShapeDtypeStruct((B, N, Cacc), jnp.float32))
        out_specs.append(per_b((N, Cacc)))
    return pl.pallas_call(
        functools.partial(_perm_body, acc_mode),
        grid=(B,),
        in_specs=in_specs,
        out_specs=out_specs,
        out_shape=out_shape,
        scratch_shapes=[pltpu.VMEM((2, Cout), jnp.float32)],
    )(*args)


def _conv_body(has_acc, raw_ref, stats_ref, w_ref, b_ref, *rest):
    if has_acc:
        (acc_ref, out_ref, stats_out_ref, sums_ref) = rest
    else:
        (out_ref, stats_out_ref, sums_ref) = rest
    i = pl.program_id(0)
    B = pl.num_programs(0)
    _, N, _ = raw_ref.shape
    f = _bn_relu_from_stats(raw_ref[0], stats_ref)
    r = jnp.dot(f, w_ref[:, :], preferred_element_type=jnp.float32) + b_ref[0:1, :]
    if has_acc:
        r = r + acc_ref[0]
    out_ref[0] = r
    _accum_stats(i, B * N, r, sums_ref, stats_out_ref, B)


def _conv_stage(raw_prev, stats_prev, W, bvec, acc_in=None):
    B, N, Cin = raw_prev.shape
    Cout = W.shape[1]
    has_acc = acc_in is not None
    full = lambda shape: pl.BlockSpec(shape, lambda b: (0,) * len(shape))
    per_b = lambda shape: pl.BlockSpec((1,) + shape, lambda b: (b,) + (0,) * len(shape))
    in_specs = [per_b((N, Cin)), full((2, Cin)), full((Cin, Cout)), full((1, Cout))]
    args = [raw_prev, stats_prev, W, bvec]
    if has_acc:
        in_specs.append(per_b((N, Cout)))
        args.append(acc_in)
    return pl.pallas_call(
        functools.partial(_conv_body, has_acc),
        grid=(B,),
        in_specs=in_specs,
        out_specs=[per_b((N, Cout)), full((2, Cout))],
        out_shape=[
            jax.ShapeDtypeStruct((B, N, Cout), jnp.float32),
            jax.ShapeDtypeStruct((2, Cout), jnp.float32),
        ],
        scratch_shapes=[pltpu.VMEM((2, Cout), jnp.float32)],
    )(*args)


def _tail_body(raw_ref, stats_ref, w8_ref, b8_ref, w9_ref, b9_ref, out_ref):
    B, N, C = raw_ref.shape
    x = raw_ref[:, :, :].reshape(B * N, C)
    f7 = _bn_relu_from_stats(x, stats_ref)
    r8 = jnp.dot(f7, w8_ref[:, :], preferred_element_type=jnp.float32) + b8_ref[0:1, :]
    m8 = jnp.mean(r8, axis=0, keepdims=True)
    v8 = jnp.mean((r8 - m8) ** 2, axis=0, keepdims=True)
    f8 = jnp.maximum((r8 - m8) / jnp.sqrt(v8 + EPS), 0.0)
    r9 = jnp.dot(f8, w9_ref[:, :], preferred_element_type=jnp.float32) + b9_ref[0:1, :]
    m9 = jnp.mean(r9, axis=0, keepdims=True)
    v9 = jnp.mean((r9 - m9) ** 2, axis=0, keepdims=True)
    y = jnp.maximum((r9 - m9) / jnp.sqrt(v9 + EPS), 0.0)
    out_ref[:, :, :] = y.reshape(B, N, -1)


def _tail_stage(raw7, stats7, W8, b8, W9, b9):
    B, N, _ = raw7.shape
    return pl.pallas_call(
        _tail_body,
        out_shape=jax.ShapeDtypeStruct((B, N, W9.shape[1]), jnp.float32),
    )(raw7, stats7, W8, b8, W9, b9)


def kernel(partial_cloud, W1, b1, W2, b2, W3, b3, W4, b4, W5, b5, W6, b6,
           W7, b7, W8, b8, W9, b9):
    B, N, _ = partial_cloud.shape
    row = lambda v: v.reshape(1, -1)

    raw1, stats1, keys = _stage1(partial_cloud, W1, row(b1))
    keys = keys.reshape(len(SCALES), B, 1, N)

    raw2, stats2 = _perm_stage(raw1, stats1, keys[0], W2, row(b2))
    offs = [0, 64, 192, 448, 704, 960]
    raw3, stats3, acc = _perm_stage(raw2, stats2, keys[1], W3, row(b3),
                                    None, W7[offs[0]:offs[1]])
    raw4, stats4, acc = _perm_stage(raw3, stats3, keys[2], W4, row(b4),
                                    acc, W7[offs[1]:offs[2]])
    raw5, stats5, acc = _perm_stage(raw4, stats4, keys[3], W5, row(b5),
                                    acc, W7[offs[2]:offs[3]])
    raw6, stats6, acc = _perm_stage(raw5, stats5, keys[4], W6, row(b6),
                                    acc, W7[offs[3]:offs[4]])
    raw7, stats7 = _conv_stage(raw6, stats6, W7[offs[4]:offs[5]], row(b7), acc)
    out = _tail_stage(raw7, stats7, W8, row(b8), W9, row(b9))
    return jnp.transpose(out, (0, 2, 1))


# speedup vs baseline: 7.5146x; 1.0878x over previous
import functools

import jax
import jax.numpy as jnp
from jax.experimental import pallas as pl
from jax.experimental.pallas import tpu as pltpu

M_LATTICE = 8192
P1, P2, P3 = 73856093, 19349663, 83492791
SCALES = (64.0, 32.0, 16.0, 8.0, 4.0)
EPS = 1e-5


def _bn_relu_from_stats(x, stats_ref):
    mean = stats_ref[0:1, :]
    var = stats_ref[1:2, :]
    return jnp.maximum((x - mean) / jnp.sqrt(var + EPS), 0.0)


def _accum_stats(i, n_total, r, sums_ref, stats_out_ref, num_programs):
    @pl.when(i == 0)
    def _init():
        sums_ref[:, :] = jnp.zeros_like(sums_ref)

    sums_ref[0:1, :] += jnp.sum(r, axis=0, keepdims=True)
    sums_ref[1:2, :] += jnp.sum(r * r, axis=0, keepdims=True)

    @pl.when(i == num_programs - 1)
    def _emit():
        m = sums_ref[0:1, :] / n_total
        v = sums_ref[1:2, :] / n_total - m * m
        stats_out_ref[0:1, :] = m
        stats_out_ref[1:2, :] = v


def _k1_body(pc_ref, pct_ref, w_ref, b_ref, raw_ref, stats_ref, keys_ref):
    B, N, _ = pc_ref.shape
    xr = pc_ref[:, :, :].reshape(B * N, 3)
    r = jnp.dot(xr, w_ref[:, :], preferred_element_type=jnp.float32) + b_ref[0:1, :]
    raw_ref[:, :, :] = r.reshape(B, N, -1)
    m = jnp.mean(r, axis=0, keepdims=True)
    v = jnp.mean((r - m) ** 2, axis=0, keepdims=True)
    stats_ref[0:1, :] = m
    stats_ref[1:2, :] = v
    xt = pct_ref[:, :]
    for i, s in enumerate(SCALES):
        ip = jnp.floor(xt * s).astype(jnp.int32)
        h = ip[0:1, :] * P1 + ip[1:2, :] * P2 + ip[2:3, :] * P3
        keys_ref[i, :, :] = jnp.bitwise_and(h, M_LATTICE - 1)


def _stage1(pc, W1, b1):
    B, N, _ = pc.shape
    C = W1.shape[1]
    pct = pc.reshape(B * N, 3).T
    return pl.pallas_call(
        _k1_body,
        out_shape=[
            jax.ShapeDtypeStruct((B, N, C), jnp.float32),
            jax.ShapeDtypeStruct((2, C), jnp.float32),
            jax.ShapeDtypeStruct((len(SCALES), 1, B * N), jnp.int32),
        ],
    )(pc, pct, W1, b1)


def _perm_body(emit_f, raw_ref, stats_ref, key_ref, w_ref, b_ref, *rest):
    if emit_f:
        (out_ref, stats_out_ref, f_out_ref, sums_ref) = rest
    else:
        (out_ref, stats_out_ref, sums_ref) = rest
    i = pl.program_id(0)
    B = pl.num_programs(0)
    _, N, Cin = raw_ref.shape
    f = _bn_relu_from_stats(raw_ref[0], stats_ref)
    k = key_ref[0]
    A = (k.reshape(N, 1) == k.reshape(1, N)).astype(jnp.bfloat16)
    f_hi = f.astype(jnp.bfloat16)
    f_lo = (f - f_hi.astype(jnp.float32)).astype(jnp.bfloat16)
    if Cin % 128 != 0:
        f_hi = jnp.concatenate([f_hi, jnp.ones((N, 1), jnp.bfloat16)], axis=1)
        f_lo = jnp.concatenate([f_lo, jnp.zeros((N, 1), jnp.bfloat16)], axis=1)
        s_aug = (jnp.dot(A, f_hi, preferred_element_type=jnp.float32)
                 + jnp.dot(A, f_lo, preferred_element_type=jnp.float32))
        s = s_aug[:, :Cin]
        cnt = s_aug[:, Cin:Cin + 1]
    else:
        s = (jnp.dot(A, f_hi, preferred_element_type=jnp.float32)
             + jnp.dot(A, f_lo, preferred_element_type=jnp.float32))
        cnt = jnp.dot(A, jnp.ones((N, 1), jnp.bfloat16),
                      preferred_element_type=jnp.float32)
    sliced = s / cnt
    r = jnp.dot(sliced, w_ref[:, :], preferred_element_type=jnp.float32) + b_ref[0:1, :]
    out_ref[0] = r
    if emit_f:
        f_out_ref[0] = f
    _accum_stats(i, B * N, r, sums_ref, stats_out_ref, B)


def _perm_stage(raw_prev, stats_prev, keys_l, W, bvec, emit_f=False):
    B, N, Cin = raw_prev.shape
    Cout = W.shape[1]
    full = lambda shape: pl.BlockSpec(shape, lambda b: (0,) * len(shape))
    per_b = lambda shape: pl.BlockSpec((1,) + shape, lambda b: (b,) + (0,) * len(shape))
    in_specs = [
        per_b((N, Cin)),
        full((2, Cin)),
        per_b((1, N)),
        full((Cin, Cout)),
        full((1, Cout)),
    ]
    out_shape = [
        jax.ShapeDtypeStruct((B, N, Cout), jnp.float32),
        jax.ShapeDtypeStruct((2, Cout), jnp.float32),
    ]
    out_specs = [per_b((N, Cout)), full((2, Cout))]
    if emit_f:
        out_shape.append(jax.ShapeDtypeStruct((B, N, Cin), jnp.float32))
        out_specs.append(per_b((N, Cin)))
    return pl.pallas_call(
        functools.partial(_perm_body, emit_f),
        grid=(B,),
        in_specs=in_specs,
        out_specs=out_specs,
        out_shape=out_shape,
        scratch_shapes=[pltpu.VMEM((2, Cout), jnp.float32)],
    )(raw_prev, stats_prev, keys_l, W, bvec)


def _conv7_body(raw_ref, stats_ref, f2_ref, f3_ref, f4_ref, f5_ref,
                w2_ref, w3_ref, w4_ref, w5_ref, w6_ref, b_ref,
                out_ref, stats_out_ref, sums_ref):
    i = pl.program_id(0)
    B = pl.num_programs(0)
    _, N, _ = raw_ref.shape
    f6 = _bn_relu_from_stats(raw_ref[0], stats_ref)
    r = jnp.dot(f2_ref[0], w2_ref[:, :], preferred_element_type=jnp.float32)
    r = r + jnp.dot(f3_ref[0], w3_ref[:, :], preferred_element_type=jnp.float32)
    r = r + jnp.dot(f4_ref[0], w4_ref[:, :], preferred_element_type=jnp.float32)
    r = r + jnp.dot(f5_ref[0], w5_ref[:, :], preferred_element_type=jnp.float32)
    r = r + jnp.dot(f6, w6_ref[:, :], preferred_element_type=jnp.float32)
    r = r + b_ref[0:1, :]
    out_ref[0] = r
    _accum_stats(i, B * N, r, sums_ref, stats_out_ref, B)


def _conv7_stage(raw6, stats6, fs, W7cs, bvec):
    B, N, _ = raw6.shape
    Cout = W7cs[0].shape[1]
    full = lambda shape: pl.BlockSpec(shape, lambda b: (0,) * len(shape))
    per_b = lambda shape: pl.BlockSpec((1,) + shape, lambda b: (b,) + (0,) * len(shape))
    in_specs = ([per_b((N, raw6.shape[2])), full((2, raw6.shape[2]))]
                + [per_b((N, f.shape[2])) for f in fs]
                + [full(w.shape) for w in W7cs]
                + [full((1, Cout))])
    return pl.pallas_call(
        _conv7_body,
        grid=(B,),
        in_specs=in_specs,
        out_specs=[per_b((N, Cout)), full((2, Cout))],
        out_shape=[
            jax.ShapeDtypeStruct((B, N, Cout), jnp.float32),
            jax.ShapeDtypeStruct((2, Cout), jnp.float32),
        ],
        scratch_shapes=[pltpu.VMEM((2, Cout), jnp.float32)],
    )(raw6, stats6, *fs, *W7cs, bvec)


def _tail_body(raw_ref, stats_ref, w8_ref, b8_ref, w9_ref, b9_ref, out_ref):
    B, N, C = raw_ref.shape
    x = raw_ref[:, :, :].reshape(B * N, C)
    f7 = _bn_relu_from_stats(x, stats_ref)
    r8 = jnp.dot(f7, w8_ref[:, :], preferred_element_type=jnp.float32) + b8_ref[0:1, :]
    m8 = jnp.mean(r8, axis=0, keepdims=True)
    v8 = jnp.mean((r8 - m8) ** 2, axis=0, keepdims=True)
    f8 = jnp.maximum((r8 - m8) / jnp.sqrt(v8 + EPS), 0.0)
    r9 = jnp.dot(f8, w9_ref[:, :], preferred_element_type=jnp.float32) + b9_ref[0:1, :]
    m9 = jnp.mean(r9, axis=0, keepdims=True)
    v9 = jnp.mean((r9 - m9) ** 2, axis=0, keepdims=True)
    y = jnp.maximum((r9 - m9) / jnp.sqrt(v9 + EPS), 0.0)
    out_ref[:, :, :] = y.reshape(B, N, -1)


def _tail_stage(raw7, stats7, W8, b8, W9, b9):
    B, N, _ = raw7.shape
    return pl.pallas_call(
        _tail_body,
        out_shape=jax.ShapeDtypeStruct((B, N, W9.shape[1]), jnp.float32),
    )(raw7, stats7, W8, b8, W9, b9)


def kernel(partial_cloud, W1, b1, W2, b2, W3, b3, W4, b4, W5, b5, W6, b6,
           W7, b7, W8, b8, W9, b9):
    B, N, _ = partial_cloud.shape
    row = lambda v: v.reshape(1, -1)

    raw1, stats1, keys = _stage1(partial_cloud, W1, row(b1))
    keys = keys.reshape(len(SCALES), B, 1, N)

    raw2, stats2 = _perm_stage(raw1, stats1, keys[0], W2, row(b2))
    raw3, stats3, f2 = _perm_stage(raw2, stats2, keys[1], W3, row(b3), emit_f=True)
    raw4, stats4, f3 = _perm_stage(raw3, stats3, keys[2], W4, row(b4), emit_f=True)
    raw5, stats5, f4 = _perm_stage(raw4, stats4, keys[3], W5, row(b5), emit_f=True)
    raw6, stats6, f5 = _perm_stage(raw5, stats5, keys[4], W6, row(b6), emit_f=True)
    offs = [0, 64, 192, 448, 704, 960]
    W7cs = [W7[offs[j]:offs[j + 1]] for j in range(5)]
    raw7, stats7 = _conv7_stage(raw6, stats6, [f2, f3, f4, f5], W7cs, row(b7))
    out = _tail_stage(raw7, stats7, W8, row(b8), W9, row(b9))
    return jnp.transpose(out, (0, 2, 1))


# speedup vs baseline: 8.1136x; 1.0797x over previous
import functools

import jax
import jax.numpy as jnp
from jax.experimental import pallas as pl
from jax.experimental.pallas import tpu as pltpu

M_LATTICE = 8192
P1, P2, P3 = 73856093, 19349663, 83492791
SCALES = (64.0, 32.0, 16.0, 8.0, 4.0)
EPS = 1e-5


def _bn_relu_from_stats(x, stats_ref):
    mean = stats_ref[0:1, :]
    var = stats_ref[1:2, :]
    return jnp.maximum((x - mean) / jnp.sqrt(var + EPS), 0.0)


def _accum_stats(i, n_total, r, sums_ref, stats_out_ref, num_programs):
    @pl.when(i == 0)
    def _init():
        sums_ref[:, :] = jnp.zeros_like(sums_ref)

    sums_ref[0:1, :] += jnp.sum(r, axis=0, keepdims=True)
    sums_ref[1:2, :] += jnp.sum(r * r, axis=0, keepdims=True)

    @pl.when(i == num_programs - 1)
    def _emit():
        m = sums_ref[0:1, :] / n_total
        v = sums_ref[1:2, :] / n_total - m * m
        stats_out_ref[0:1, :] = m
        stats_out_ref[1:2, :] = v


def _k1_body(pc_ref, pct_ref, w_ref, b_ref, raw_ref, stats_ref, keys_ref):
    B, N, _ = pc_ref.shape
    xr = pc_ref[:, :, :].reshape(B * N, 3)
    r = jnp.dot(xr, w_ref[:, :], preferred_element_type=jnp.float32) + b_ref[0:1, :]
    raw_ref[:, :, :] = r.reshape(B, N, -1)
    m = jnp.mean(r, axis=0, keepdims=True)
    v = jnp.mean((r - m) ** 2, axis=0, keepdims=True)
    stats_ref[0:1, :] = m
    stats_ref[1:2, :] = v
    xt = pct_ref[:, :]
    for i, s in enumerate(SCALES):
        ip = jnp.floor(xt * s).astype(jnp.int32)
        h = ip[0:1, :] * P1 + ip[1:2, :] * P2 + ip[2:3, :] * P3
        keys_ref[i, :, :] = jnp.bitwise_and(h, M_LATTICE - 1)


def _stage1(pc, W1, b1):
    B, N, _ = pc.shape
    C = W1.shape[1]
    pct = pc.reshape(B * N, 3).T
    return pl.pallas_call(
        _k1_body,
        out_shape=[
            jax.ShapeDtypeStruct((B, N, C), jnp.float32),
            jax.ShapeDtypeStruct((2, C), jnp.float32),
            jax.ShapeDtypeStruct((len(SCALES), 1, B * N), jnp.int32),
        ],
    )(pc, pct, W1, b1)


def _perm_body(emit_f, raw_ref, stats_ref, key_ref, w_ref, b_ref, *rest):
    if emit_f:
        (out_ref, stats_out_ref, f_out_ref, sums_ref) = rest
    else:
        (out_ref, stats_out_ref, sums_ref) = rest
    i = pl.program_id(0)
    B = pl.num_programs(0)
    _, N, Cin = raw_ref.shape
    f = _bn_relu_from_stats(raw_ref[0], stats_ref)
    k = key_ref[0]
    A = (k.reshape(N, 1) == k.reshape(1, N)).astype(jnp.bfloat16)
    f_hi = f.astype(jnp.bfloat16)
    f_lo = (f - f_hi.astype(jnp.float32)).astype(jnp.bfloat16)
    if Cin % 128 != 0:
        f_hi = jnp.concatenate([f_hi, jnp.ones((N, 1), jnp.bfloat16)], axis=1)
        f_lo = jnp.concatenate([f_lo, jnp.zeros((N, 1), jnp.bfloat16)], axis=1)
        s_aug = (jnp.dot(A, f_hi, preferred_element_type=jnp.float32)
                 + jnp.dot(A, f_lo, preferred_element_type=jnp.float32))
        s = s_aug[:, :Cin]
        cnt = s_aug[:, Cin:Cin + 1]
    else:
        s = (jnp.dot(A, f_hi, preferred_element_type=jnp.float32)
             + jnp.dot(A, f_lo, preferred_element_type=jnp.float32))
        cnt = jnp.sum((k.reshape(N, 1) == k.reshape(1, N)).astype(jnp.float32),
                      axis=1, keepdims=True)
    sliced = s / cnt
    r = jnp.dot(sliced, w_ref[:, :], preferred_element_type=jnp.float32) + b_ref[0:1, :]
    out_ref[0] = r
    if emit_f:
        f_out_ref[0] = f_hi[:, :Cin] if Cin % 128 != 0 else f_hi
    _accum_stats(i, B * N, r, sums_ref, stats_out_ref, B)


def _perm_stage(raw_prev, stats_prev, keys_l, W, bvec, emit_f=False):
    B, N, Cin = raw_prev.shape
    Cout = W.shape[1]
    full = lambda shape: pl.BlockSpec(shape, lambda b: (0,) * len(shape))
    per_b = lambda shape: pl.BlockSpec((1,) + shape, lambda b: (b,) + (0,) * len(shape))
    in_specs = [
        per_b((N, Cin)),
        full((2, Cin)),
        per_b((1, N)),
        full((Cin, Cout)),
        full((1, Cout)),
    ]
    out_shape = [
        jax.ShapeDtypeStruct((B, N, Cout), jnp.float32),
        jax.ShapeDtypeStruct((2, Cout), jnp.float32),
    ]
    out_specs = [per_b((N, Cout)), full((2, Cout))]
    if emit_f:
        out_shape.append(jax.ShapeDtypeStruct((B, N, Cin), jnp.bfloat16))
        out_specs.append(per_b((N, Cin)))
    return pl.pallas_call(
        functools.partial(_perm_body, emit_f),
        grid=(B,),
        in_specs=in_specs,
        out_specs=out_specs,
        out_shape=out_shape,
        scratch_shapes=[pltpu.VMEM((2, Cout), jnp.float32)],
    )(raw_prev, stats_prev, keys_l, W, bvec)


def _conv7_body(raw_ref, stats_ref, f2_ref, f3_ref, f4_ref, f5_ref,
                w2_ref, w3_ref, w4_ref, w5_ref, w6_ref, b_ref,
                out_ref, stats_out_ref, sums_ref):
    i = pl.program_id(0)
    B = pl.num_programs(0)
    _, N, _ = raw_ref.shape
    f6 = _bn_relu_from_stats(raw_ref[0], stats_ref)
    r = jnp.dot(f2_ref[0], w2_ref[:, :], preferred_element_type=jnp.float32)
    r = r + jnp.dot(f3_ref[0], w3_ref[:, :], preferred_element_type=jnp.float32)
    r = r + jnp.dot(f4_ref[0], w4_ref[:, :], preferred_element_type=jnp.float32)
    r = r + jnp.dot(f5_ref[0], w5_ref[:, :], preferred_element_type=jnp.float32)
    r = r + jnp.dot(f6, w6_ref[:, :], preferred_element_type=jnp.float32)
    r = r + b_ref[0:1, :]
    out_ref[0] = r
    _accum_stats(i, B * N, r, sums_ref, stats_out_ref, B)


def _conv7_stage(raw6, stats6, fs, W7cs, bvec):
    B, N, _ = raw6.shape
    Cout = W7cs[0].shape[1]
    full = lambda shape: pl.BlockSpec(shape, lambda b: (0,) * len(shape))
    per_b = lambda shape: pl.BlockSpec((1,) + shape, lambda b: (b,) + (0,) * len(shape))
    in_specs = ([per_b((N, raw6.shape[2])), full((2, raw6.shape[2]))]
                + [per_b((N, f.shape[2])) for f in fs]
                + [full(w.shape) for w in W7cs]
                + [full((1, Cout))])
    return pl.pallas_call(
        _conv7_body,
        grid=(B,),
        in_specs=in_specs,
        out_specs=[per_b((N, Cout)), full((2, Cout))],
        out_shape=[
            jax.ShapeDtypeStruct((B, N, Cout), jnp.float32),
            jax.ShapeDtypeStruct((2, Cout), jnp.float32),
        ],
        scratch_shapes=[pltpu.VMEM((2, Cout), jnp.float32)],
    )(raw6, stats6, *fs, *W7cs, bvec)


def _tail_body(raw_ref, stats_ref, w8_ref, b8_ref, w9_ref, b9_ref, out_ref):
    B, N, C = raw_ref.shape
    x = raw_ref[:, :, :].reshape(B * N, C)
    f7 = _bn_relu_from_stats(x, stats_ref)
    r8 = jnp.dot(f7, w8_ref[:, :], preferred_element_type=jnp.float32) + b8_ref[0:1, :]
    m8 = jnp.mean(r8, axis=0, keepdims=True)
    v8 = jnp.mean((r8 - m8) ** 2, axis=0, keepdims=True)
    f8 = jnp.maximum((r8 - m8) / jnp.sqrt(v8 + EPS), 0.0)
    r9 = jnp.dot(f8, w9_ref[:, :], preferred_element_type=jnp.float32) + b9_ref[0:1, :]
    m9 = jnp.mean(r9, axis=0, keepdims=True)
    v9 = jnp.mean((r9 - m9) ** 2, axis=0, keepdims=True)
    y = jnp.maximum((r9 - m9) / jnp.sqrt(v9 + EPS), 0.0)
    out_ref[:, :, :] = y.reshape(B, N, -1)


def _tail_stage(raw7, stats7, W8, b8, W9, b9):
    B, N, _ = raw7.shape
    return pl.pallas_call(
        _tail_body,
        out_shape=jax.ShapeDtypeStruct((B, N, W9.shape[1]), jnp.float32),
    )(raw7, stats7, W8, b8, W9, b9)


def kernel(partial_cloud, W1, b1, W2, b2, W3, b3, W4, b4, W5, b5, W6, b6,
           W7, b7, W8, b8, W9, b9):
    B, N, _ = partial_cloud.shape
    row = lambda v: v.reshape(1, -1)

    raw1, stats1, keys = _stage1(partial_cloud, W1, row(b1))
    keys = keys.reshape(len(SCALES), B, 1, N)

    raw2, stats2 = _perm_stage(raw1, stats1, keys[0], W2, row(b2))
    raw3, stats3, f2 = _perm_stage(raw2, stats2, keys[1], W3, row(b3), emit_f=True)
    raw4, stats4, f3 = _perm_stage(raw3, stats3, keys[2], W4, row(b4), emit_f=True)
    raw5, stats5, f4 = _perm_stage(raw4, stats4, keys[3], W5, row(b5), emit_f=True)
    raw6, stats6, f5 = _perm_stage(raw5, stats5, keys[4], W6, row(b6), emit_f=True)
    offs = [0, 64, 192, 448, 704, 960]
    W7cs = [W7[offs[j]:offs[j + 1]] for j in range(5)]
    raw7, stats7 = _conv7_stage(raw6, stats6, [f2, f3, f4, f5], W7cs, row(b7))
    out = _tail_stage(raw7, stats7, W8, row(b8), W9, row(b9))
    return jnp.transpose(out, (0, 2, 1))
